# baseline (speedup 1.0000x reference)
import functools

import jax
import jax.numpy as jnp
from jax import lax
from jax.experimental import pallas as pl
from jax.experimental.pallas import tpu as pltpu
from jax.experimental.pallas import tpu_sc as plsc

_NC = 2
_NS = 16
_C = 128
_N_PAD = 10240


def _sc_segment_sum(x, src, dst, nct):
  n_nodes, d = x.shape
  n_pad = _N_PAD
  rows_per_tile = n_pad // _NS
  assert rows_per_tile % _C == 0
  assert nct % 2 == 0

  mesh = plsc.VectorSubcoreMesh(
      core_axis_name="c", subcore_axis_name="s",
      num_cores=_NC, num_subcores=_NS)

  @functools.partial(
      pl.kernel,
      out_type=jax.ShapeDtypeStruct((_NC, n_pad, d), jnp.float32),
      mesh=mesh,
      scratch_types=[
          pltpu.VMEM_SHARED((n_pad, d), jnp.float32),
          pltpu.VMEM((_C,), jnp.int32),
          pltpu.VMEM((_C,), jnp.int32),
          pltpu.VMEM((_C,), jnp.int32),
          pltpu.VMEM((_C,), jnp.int32),
          pltpu.VMEM((_C, d), jnp.float32),
          pltpu.VMEM((_C, d), jnp.float32),
          pltpu.SemaphoreType.DMA,
      ],
  )
  def k(x_hbm, src_hbm, dst_hbm, out_hbm, acc,
        sidx0, sidx1, didx0, didx1, rows0, rows1, gsem):
    cid = lax.axis_index("c")
    tid = lax.axis_index("s")
    w = cid * _NS + tid
    e0 = w * nct * _C
    sidx = (sidx0, sidx1)
    didx = (didx0, didx1)
    rows = (rows0, rows1)

    def idx_load(j, b):
      off = pl.multiple_of(e0 + j * _C, _C)
      pltpu.sync_copy(src_hbm.at[pl.ds(off, _C)], sidx[b])
      pltpu.sync_copy(dst_hbm.at[pl.ds(off, _C)], didx[b])

    def zrow(i, _):
      for jj in range(d // 16):
        rows1[i, pl.ds(jj * 16, 16)] = jnp.zeros((16,), jnp.float32)
      return 0
    lax.fori_loop(0, _C, zrow, 0)
    r0 = tid * rows_per_tile
    for kk in range(rows_per_tile // _C):
      pltpu.sync_copy(rows1, acc.at[pl.ds(r0 + kk * _C, _C)])

    idx_load(0, 0)
    g0 = pltpu.async_copy(x_hbm.at[sidx0], rows0, gsem)
    plsc.subcore_barrier()

    def step(j, b, issue_next):
      if issue_next:
        idx_load(j + 1, 1 - b)
        pltpu.async_copy(x_hbm.at[sidx[1 - b]], rows[1 - b], gsem)
      pltpu.make_async_copy(x_hbm.at[sidx[b]], rows[b], gsem).wait()
      pltpu.sync_copy(rows[b], acc.at[didx[b]], add=True)

    def body(i, _):
      j = 2 * i
      step(j, 0, True)
      step(j + 1, 1, True)
      return 0
    lax.fori_loop(0, nct // 2 - 1, body, 0)
    step(nct - 2, 0, True)
    step(nct - 1, 1, False)
    plsc.subcore_barrier()

    pltpu.sync_copy(acc.at[pl.ds(r0, rows_per_tile)],
                    out_hbm.at[cid, pl.ds(r0, rows_per_tile)])

  return k(x, src, dst)


def _tc_body(p_ref, x_ref, wm_ref, ws_ref, b_ref, o_ref):
  agg = p_ref[0] + p_ref[1]
  h = jnp.dot(agg, wm_ref[...], preferred_element_type=jnp.float32)
  h = h + jnp.dot(x_ref[...], ws_ref[...], preferred_element_type=jnp.float32)
  o_ref[...] = jnp.maximum(h + b_ref[...], 0.0)


def _tc_dense(partials, x, w_msg, w_self, b2):
  n, d = x.shape
  br = 2000
  grid = (n // br,)
  return pl.pallas_call(
      _tc_body,
      grid=grid,
      in_specs=[
          pl.BlockSpec((_NC, br, d), lambda i: (0, i, 0)),
          pl.BlockSpec((br, d), lambda i: (i, 0)),
          pl.BlockSpec((d, d), lambda i: (0, 0)),
          pl.BlockSpec((d, d), lambda i: (0, 0)),
          pl.BlockSpec((1, d), lambda i: (0, 0)),
      ],
      out_specs=pl.BlockSpec((br, d), lambda i: (i, 0)),
      out_shape=jax.ShapeDtypeStruct((n, d), jnp.float32),
  )(partials, x, w_msg, w_self, b2)


@jax.jit
def kernel(x, edge_index, W_msg, W_self, b):
  ei = edge_index.astype(jnp.int32)
  src = ei[0]
  dst = ei[1]
  n_edges = src.shape[0]
  nw = _NC * _NS
  nct = -(-n_edges // (nw * _C))
  nct = -(-nct // 8) * 8
  pad = nw * nct * _C - n_edges
  if pad:
    src = jnp.concatenate([src, jnp.zeros((pad,), jnp.int32)])
    dst = jnp.concatenate([dst, jnp.full((pad,), _N_PAD - 1, jnp.int32)])
  partials = _sc_segment_sum(x, src, dst, nct)
  return _tc_dense(partials, x, W_msg, W_self, b.reshape(1, -1))


# speedup vs baseline: 3.0533x; 3.0533x over previous
import functools

import jax
import jax.numpy as jnp
from jax import lax
from jax.experimental import pallas as pl
from jax.experimental.pallas import tpu as pltpu
from jax.experimental.pallas import tpu_sc as plsc

_NC = 2
_NS = 16
_C = 128
_N_PAD = 10240


def _sc_segment_sum(x, src, dst):
  n_nodes, d = x.shape
  n_pad = _N_PAD
  n_edges = src.shape[0]
  rows_per_tile = n_pad // _NS
  assert rows_per_tile % _C == 0
  nw = _NC * _NS
  nfull = n_edges // _C
  tail = n_edges % _C
  per, rem = divmod(nfull, nw)
  assert per % 2 == 0 and per >= 4

  mesh = plsc.VectorSubcoreMesh(
      core_axis_name="c", subcore_axis_name="s",
      num_cores=_NC, num_subcores=_NS)

  @functools.partial(
      pl.kernel,
      out_type=jax.ShapeDtypeStruct((_NC, n_pad, d), jnp.float32),
      mesh=mesh,
      scratch_types=[
          pltpu.VMEM_SHARED((n_pad, d), jnp.float32),
          pltpu.VMEM((_C,), jnp.int32),
          pltpu.VMEM((_C,), jnp.int32),
          pltpu.VMEM((_C,), jnp.int32),
          pltpu.VMEM((_C,), jnp.int32),
          pltpu.VMEM((_C, d), jnp.float32),
          pltpu.VMEM((_C, d), jnp.float32),
          pltpu.SemaphoreType.DMA,
      ],
  )
  def k(x_hbm, src_hbm, dst_hbm, out_hbm, acc,
        sidx0, sidx1, didx0, didx1, rows0, rows1, gsem):
    cid = lax.axis_index("c")
    tid = lax.axis_index("s")
    w = cid * _NS + tid
    cbase = w * per + jnp.minimum(w, rem)
    sidx = (sidx0, sidx1)
    didx = (didx0, didx1)
    rows = (rows0, rows1)

    def idx_load(j, b):
      off = pl.multiple_of((cbase + j) * _C, _C)
      pltpu.sync_copy(src_hbm.at[pl.ds(off, _C)], sidx[b])
      pltpu.sync_copy(dst_hbm.at[pl.ds(off, _C)], didx[b])

    def zrow(i, _):
      for jj in range(d // 16):
        rows1[i, pl.ds(jj * 16, 16)] = jnp.zeros((16,), jnp.float32)
      return 0
    lax.fori_loop(0, _C, zrow, 0)
    r0 = tid * rows_per_tile
    for kk in range(rows_per_tile // _C):
      pltpu.sync_copy(rows1, acc.at[pl.ds(r0 + kk * _C, _C)])

    idx_load(0, 0)
    g0 = pltpu.async_copy(x_hbm.at[sidx0], rows0, gsem)
    plsc.subcore_barrier()

    def step(j, b, issue_next):
      if issue_next:
        idx_load(j + 1, 1 - b)
        pltpu.async_copy(x_hbm.at[sidx[1 - b]], rows[1 - b], gsem)
      pltpu.make_async_copy(x_hbm.at[sidx[b]], rows[b], gsem).wait()
      pltpu.sync_copy(rows[b], acc.at[didx[b]], add=True)

    def body(i, _):
      j = 2 * i
      step(j, 0, True)
      step(j + 1, 1, True)
      return 0
    lax.fori_loop(0, per // 2 - 1, body, 0)
    step(per - 2, 0, True)
    step(per - 1, 1, False)

    if rem:
      @pl.when(w < rem)
      def _():
        idx_load(per, 0)
        pltpu.async_copy(x_hbm.at[sidx0], rows0, gsem).wait()
        pltpu.sync_copy(rows0, acc.at[didx0], add=True)

    if tail:
      @pl.when(w == 0)
      def _():
        toff = nfull * _C
        pltpu.sync_copy(src_hbm.at[pl.ds(toff, tail)], sidx0.at[pl.ds(0, tail)])
        pltpu.sync_copy(dst_hbm.at[pl.ds(toff, tail)], didx0.at[pl.ds(0, tail)])
        pltpu.async_copy(
            x_hbm.at[sidx0.at[pl.ds(0, tail)]], rows0.at[pl.ds(0, tail)],
            gsem).wait()
        pltpu.sync_copy(rows0.at[pl.ds(0, tail)],
                        acc.at[didx0.at[pl.ds(0, tail)]], add=True)
    plsc.subcore_barrier()

    pltpu.sync_copy(acc.at[pl.ds(r0, rows_per_tile)],
                    out_hbm.at[cid, pl.ds(r0, rows_per_tile)])

  return k(x, src, dst)


def _tc_body(p_ref, x_ref, wm_ref, ws_ref, b_ref, o_ref):
  agg = p_ref[0] + p_ref[1]
  h = jnp.dot(agg, wm_ref[...], preferred_element_type=jnp.float32)
  h = h + jnp.dot(x_ref[...], ws_ref[...], preferred_element_type=jnp.float32)
  o_ref[...] = jnp.maximum(h + b_ref[...], 0.0)


def _tc_dense(partials, x, w_msg, w_self, b2):
  n, d = x.shape
  br = 2000
  grid = (n // br,)
  return pl.pallas_call(
      _tc_body,
      grid=grid,
      in_specs=[
          pl.BlockSpec((_NC, br, d), lambda i: (0, i, 0)),
          pl.BlockSpec((br, d), lambda i: (i, 0)),
          pl.BlockSpec((d, d), lambda i: (0, 0)),
          pl.BlockSpec((d, d), lambda i: (0, 0)),
          pl.BlockSpec((1, d), lambda i: (0, 0)),
      ],
      out_specs=pl.BlockSpec((br, d), lambda i: (i, 0)),
      out_shape=jax.ShapeDtypeStruct((n, d), jnp.float32),
  )(partials, x, w_msg, w_self, b2)


@jax.jit
def kernel(x, edge_index, W_msg, W_self, b):
  ei = edge_index.astype(jnp.int32)
  partials = _sc_segment_sum(x, ei[0], ei[1])
  return _tc_dense(partials, x, W_msg, W_self, b.reshape(1, -1))


# speedup vs baseline: 3.9833x; 1.3046x over previous
import functools

import jax
import jax.numpy as jnp
from jax import lax
from jax.experimental import pallas as pl
from jax.experimental.pallas import tpu as pltpu
from jax.experimental.pallas import tpu_sc as plsc

_NC = 2
_NS = 16
_C = 128
_N_PAD = 10240


def _sc_segment_sum(x, src, dst):
  n_nodes, d = x.shape
  n_pad = _N_PAD
  n_edges = src.shape[0]
  rows_per_tile = n_pad // _NS
  assert rows_per_tile % _C == 0
  nw = _NC * _NS
  nfull = n_edges // _C
  tail = n_edges % _C
  per, rem = divmod(nfull, nw)
  assert per % 2 == 0 and per >= 4

  mesh = plsc.VectorSubcoreMesh(
      core_axis_name="c", subcore_axis_name="s",
      num_cores=_NC, num_subcores=_NS)

  @functools.partial(
      pl.kernel,
      out_type=jax.ShapeDtypeStruct((_NC, n_pad, d), jnp.float32),
      mesh=mesh,
      scratch_types=[
          pltpu.VMEM_SHARED((n_pad, d), jnp.float32),
          pltpu.VMEM((4, _C), jnp.int32),
          pltpu.VMEM((4, _C), jnp.int32),
          pltpu.VMEM((_C, d), jnp.float32),
          pltpu.VMEM((_C, d), jnp.float32),
          pltpu.SemaphoreType.DMA,
          pltpu.SemaphoreType.DMA,
      ],
  )
  def k(x_hbm, src_hbm, dst_hbm, out_hbm, acc, sidx, didx, rows0, rows1,
        gsem, isem):
    cid = lax.axis_index("c")
    tid = lax.axis_index("s")
    w = cid * _NS + tid
    cbase = w * per + jnp.minimum(w, rem)
    rows = (rows0, rows1)

    def idx_issue(j):
      off = pl.multiple_of((cbase + j) * _C, _C)
      pltpu.async_copy(src_hbm.at[pl.ds(off, _C)], sidx.at[j % 4], isem)
      pltpu.async_copy(dst_hbm.at[pl.ds(off, _C)], didx.at[j % 4], isem)

    def idx_wait(j):
      pltpu.make_async_copy(src_hbm.at[pl.ds(0, _C)], sidx.at[j % 4], isem).wait()
      pltpu.make_async_copy(dst_hbm.at[pl.ds(0, _C)], didx.at[j % 4], isem).wait()

    def gather_issue(j, b):
      pltpu.async_copy(x_hbm.at[sidx.at[j % 4]], rows[b], gsem)

    def gather_wait(j, b):
      pltpu.make_async_copy(x_hbm.at[sidx.at[j % 4]], rows[b], gsem).wait()

    for j in range(3):
      idx_issue(j)
    idx_wait(0)
    gather_issue(0, 0)

    def zrow(i, _):
      for jj in range(d // 16):
        rows1[i, pl.ds(jj * 16, 16)] = jnp.zeros((16,), jnp.float32)
      return 0
    lax.fori_loop(0, _C, zrow, 0)
    r0 = tid * rows_per_tile
    for kk in range(rows_per_tile // _C):
      pltpu.sync_copy(rows1, acc.at[pl.ds(r0 + kk * _C, _C)])
    plsc.subcore_barrier()

    def step(j, b, issue_idx, issue_gather):
      if issue_gather:
        idx_wait(j + 1)
        gather_issue(j + 1, 1 - b)
      gather_wait(j, b)
      if issue_idx:
        idx_issue(j + 3)
      pltpu.sync_copy(rows[b], acc.at[didx.at[j % 4]], add=True)

    def body(i, _):
      j = 2 * i
      step(j, 0, True, True)
      step(j + 1, 1, True, True)
      return 0
    lax.fori_loop(0, per // 2 - 2, body, 0)
    step(per - 4, 0, True, True)
    step(per - 3, 1, False, True)
    step(per - 2, 0, False, True)
    step(per - 1, 1, False, False)

    if rem:
      @pl.when(w < rem)
      def _():
        off = pl.multiple_of((cbase + per) * _C, _C)
        pltpu.sync_copy(src_hbm.at[pl.ds(off, _C)], sidx.at[0])
        pltpu.sync_copy(dst_hbm.at[pl.ds(off, _C)], didx.at[0])
        pltpu.async_copy(x_hbm.at[sidx.at[0]], rows0, gsem).wait()
        pltpu.sync_copy(rows0, acc.at[didx.at[0]], add=True)

    if tail:
      @pl.when(w == 0)
      def _():
        toff = nfull * _C
        pltpu.sync_copy(src_hbm.at[pl.ds(toff, tail)],
                        sidx.at[0, pl.ds(0, tail)])
        pltpu.sync_copy(dst_hbm.at[pl.ds(toff, tail)],
                        didx.at[0, pl.ds(0, tail)])
        pltpu.async_copy(
            x_hbm.at[sidx.at[0, pl.ds(0, tail)]], rows0.at[pl.ds(0, tail)],
            gsem).wait()
        pltpu.sync_copy(rows0.at[pl.ds(0, tail)],
                        acc.at[didx.at[0, pl.ds(0, tail)]], add=True)
    plsc.subcore_barrier()

    pltpu.sync_copy(acc.at[pl.ds(r0, rows_per_tile)],
                    out_hbm.at[cid, pl.ds(r0, rows_per_tile)])

  return k(x, src, dst)


def _tc_body(p_ref, x_ref, wm_ref, ws_ref, b_ref, o_ref):
  agg = p_ref[0] + p_ref[1]
  h = jnp.dot(agg, wm_ref[...], preferred_element_type=jnp.float32)
  h = h + jnp.dot(x_ref[...], ws_ref[...], preferred_element_type=jnp.float32)
  o_ref[...] = jnp.maximum(h + b_ref[...], 0.0)


def _tc_dense(partials, x, w_msg, w_self, b2):
  n, d = x.shape
  br = 2000
  grid = (n // br,)
  return pl.pallas_call(
      _tc_body,
      grid=grid,
      in_specs=[
          pl.BlockSpec((_NC, br, d), lambda i: (0, i, 0)),
          pl.BlockSpec((br, d), lambda i: (i, 0)),
          pl.BlockSpec((d, d), lambda i: (0, 0)),
          pl.BlockSpec((d, d), lambda i: (0, 0)),
          pl.BlockSpec((1, d), lambda i: (0, 0)),
      ],
      out_specs=pl.BlockSpec((br, d), lambda i: (i, 0)),
      out_shape=jax.ShapeDtypeStruct((n, d), jnp.float32),
  )(partials, x, w_msg, w_self, b2)


@jax.jit
def kernel(x, edge_index, W_msg, W_self, b):
  ei = edge_index.astype(jnp.int32)
  partials = _sc_segment_sum(x, ei[0], ei[1])
  return _tc_dense(partials, x, W_msg, W_self, b.reshape(1, -1))


# speedup vs baseline: 4.2904x; 1.0771x over previous
import functools

import jax
import jax.numpy as jnp
from jax import lax
from jax.experimental import pallas as pl
from jax.experimental.pallas import tpu as pltpu
from jax.experimental.pallas import tpu_sc as plsc

_NC = 2
_NS = 16
_C = 128
_N_PAD = 10240


def _sc_segment_sum(x, ei_flat, n_edges):
  n_nodes, d = x.shape
  n_pad = _N_PAD
  assert n_edges % 8 == 0
  rows_per_tile = n_pad // _NS
  assert rows_per_tile % _C == 0
  nw = _NC * _NS
  nfull = n_edges // _C
  tail = n_edges % _C
  per, rem = divmod(nfull, nw)
  assert per % 2 == 0 and per >= 4

  mesh = plsc.VectorSubcoreMesh(
      core_axis_name="c", subcore_axis_name="s",
      num_cores=_NC, num_subcores=_NS)

  @functools.partial(
      pl.kernel,
      out_type=jax.ShapeDtypeStruct((_NC, n_pad, d), jnp.float32),
      mesh=mesh,
      scratch_types=[
          pltpu.VMEM_SHARED((n_pad, d), jnp.float32),
          pltpu.VMEM((4, _C), jnp.int32),
          pltpu.VMEM((4, _C), jnp.int32),
          pltpu.VMEM((_C, d), jnp.float32),
          pltpu.VMEM((_C, d), jnp.float32),
          pltpu.SemaphoreType.DMA,
          pltpu.SemaphoreType.DMA,
      ],
  )
  def k(x_hbm, ei_hbm, out_hbm, acc, sidx, didx, rows0, rows1,
        gsem, isem):
    cid = lax.axis_index("c")
    tid = lax.axis_index("s")
    w = cid * _NS + tid
    cbase = w * per + jnp.minimum(w, rem)
    rows = (rows0, rows1)

    def idx_issue(j):
      off = pl.multiple_of((cbase + j) * _C, _C)
      pltpu.async_copy(ei_hbm.at[pl.ds(off, _C)], sidx.at[j % 4], isem)
      pltpu.async_copy(ei_hbm.at[pl.ds(n_edges + off, _C)], didx.at[j % 4], isem)

    def idx_wait(j):
      pltpu.make_async_copy(ei_hbm.at[pl.ds(0, _C)], sidx.at[j % 4], isem).wait()
      pltpu.make_async_copy(ei_hbm.at[pl.ds(0, _C)], didx.at[j % 4], isem).wait()

    def gather_issue(j, b):
      pltpu.async_copy(x_hbm.at[sidx.at[j % 4]], rows[b], gsem)

    def gather_wait(j, b):
      pltpu.make_async_copy(x_hbm.at[sidx.at[j % 4]], rows[b], gsem).wait()

    for j in range(3):
      idx_issue(j)
    idx_wait(0)
    gather_issue(0, 0)

    def zrow(i, _):
      for jj in range(d // 16):
        rows1[i, pl.ds(jj * 16, 16)] = jnp.zeros((16,), jnp.float32)
      return 0
    lax.fori_loop(0, _C, zrow, 0)
    r0 = tid * rows_per_tile
    for kk in range(rows_per_tile // _C):
      pltpu.sync_copy(rows1, acc.at[pl.ds(r0 + kk * _C, _C)])
    plsc.subcore_barrier()

    def step(j, b, issue_idx, issue_gather):
      if issue_gather:
        idx_wait(j + 1)
        gather_issue(j + 1, 1 - b)
      gather_wait(j, b)
      if issue_idx:
        idx_issue(j + 3)
      pltpu.sync_copy(rows[b], acc.at[didx.at[j % 4]], add=True)

    def body(i, _):
      j = 2 * i
      step(j, 0, True, True)
      step(j + 1, 1, True, True)
      return 0
    lax.fori_loop(0, per // 2 - 2, body, 0)
    step(per - 4, 0, True, True)
    step(per - 3, 1, False, True)
    step(per - 2, 0, False, True)
    step(per - 1, 1, False, False)

    if rem:
      @pl.when(w < rem)
      def _():
        off = pl.multiple_of((cbase + per) * _C, _C)
        pltpu.sync_copy(ei_hbm.at[pl.ds(off, _C)], sidx.at[0])
        pltpu.sync_copy(ei_hbm.at[pl.ds(n_edges + off, _C)], didx.at[0])
        pltpu.async_copy(x_hbm.at[sidx.at[0]], rows0, gsem).wait()
        pltpu.sync_copy(rows0, acc.at[didx.at[0]], add=True)

    if tail:
      @pl.when(w == 0)
      def _():
        toff = nfull * _C
        pltpu.sync_copy(ei_hbm.at[pl.ds(toff, tail)],
                        sidx.at[0, pl.ds(0, tail)])
        pltpu.sync_copy(ei_hbm.at[pl.ds(n_edges + toff, tail)],
                        didx.at[0, pl.ds(0, tail)])
        pltpu.async_copy(
            x_hbm.at[sidx.at[0, pl.ds(0, tail)]], rows0.at[pl.ds(0, tail)],
            gsem).wait()
        pltpu.sync_copy(rows0.at[pl.ds(0, tail)],
                        acc.at[didx.at[0, pl.ds(0, tail)]], add=True)
    plsc.subcore_barrier()

    pltpu.sync_copy(acc.at[pl.ds(r0, rows_per_tile)],
                    out_hbm.at[cid, pl.ds(r0, rows_per_tile)])

  return k(x, ei_flat)


def _tc_body(p_ref, x_ref, wm_ref, ws_ref, b_ref, o_ref):
  agg = p_ref[0] + p_ref[1]
  h = jnp.dot(agg, wm_ref[...], preferred_element_type=jnp.float32)
  h = h + jnp.dot(x_ref[...], ws_ref[...], preferred_element_type=jnp.float32)
  o_ref[...] = jnp.maximum(h + b_ref[...], 0.0)


def _tc_dense(partials, x, w_msg, w_self, b2):
  n, d = x.shape
  br = 2000
  grid = (n // br,)
  return pl.pallas_call(
      _tc_body,
      grid=grid,
      in_specs=[
          pl.BlockSpec((_NC, br, d), lambda i: (0, i, 0)),
          pl.BlockSpec((br, d), lambda i: (i, 0)),
          pl.BlockSpec((d, d), lambda i: (0, 0)),
          pl.BlockSpec((d, d), lambda i: (0, 0)),
          pl.BlockSpec((1, d), lambda i: (0, 0)),
      ],
      out_specs=pl.BlockSpec((br, d), lambda i: (i, 0)),
      out_shape=jax.ShapeDtypeStruct((n, d), jnp.float32),
  )(partials, x, w_msg, w_self, b2)


@jax.jit
def kernel(x, edge_index, W_msg, W_self, b):
  ei_flat = edge_index.astype(jnp.int32).reshape(-1)
  partials = _sc_segment_sum(x, ei_flat, edge_index.shape[1])
  return _tc_dense(partials, x, W_msg, W_self, b.reshape(1, -1))
